# baseline (speedup 1.0000x reference)
import functools

import jax
import jax.numpy as jnp
import numpy as np
from jax import lax
from jax.experimental import pallas as pl
from jax.experimental.pallas import tpu as pltpu
from jax.experimental.pallas import tpu_sc as plsc

NC = 2
NS = 16
LANES = 16
T_SC = 512
SC_CHUNK = 64
TC_CHUNK = 512


def _tc_kernel(len_ref, x_ref, wq_ref, out_ref, *, t_chunk: int, t0_base: int):
    kt = pl.program_id(1)

    @pl.when(kt == 0)
    def _init():
        out_ref[...] = jnp.zeros_like(out_ref)

    xb = x_ref[0]
    d = xb.shape[1]
    w = jnp.sum(xb * wq_ref[...].reshape(1, d), axis=1)

    t0 = t0_base + kt * t_chunk
    t_idx = jax.lax.broadcasted_iota(jnp.int32, (1, t_chunk), 1) + t0
    mask = (t_idx < len_ref[...]).astype(jnp.float32)
    a = mask * w.reshape(1, t_chunk)

    out_ref[0] += jax.lax.dot_general(
        a, xb, (((1,), (0,)), ((), ())),
        preferred_element_type=jnp.float32)


_GATHER_DNUMS = lax.GatherDimensionNumbers(
    offset_dims=(), collapsed_slice_dims=(0,), start_index_map=(0,))


def _lane_perm(v, idx):
    return lax.gather(
        v, idx[:, None], _GATHER_DNUMS, slice_sizes=(1,),
        mode=lax.GatherScatterMode.PROMISE_IN_BOUNDS)


def _lane_sum_splat(v):
    idx = lax.iota(jnp.int32, LANES)
    for k in (1, 2, 4, 8):
        v = v + _lane_perm(v, jnp.bitwise_xor(idx, k))
    return v


def _sc_body(slen_hbm, x_hbm, wq_hbm, g_hbm,
             buf0, buf1, acc, slen_v, wq_v, sem0, sem1, *, d: int, nseg: int):
    c = lax.axis_index("c")
    s = lax.axis_index("s")
    b = s
    rows = T_SC // NC
    row0 = c * rows
    nchunk = rows // SC_CHUNK
    dch = d // LANES

    pltpu.sync_copy(wq_hbm, wq_v)
    pltpu.sync_copy(slen_hbm, slen_v)

    zero = jnp.zeros((LANES,), jnp.float32)
    for j in range(nseg * dch):
        acc[pl.ds(j * LANES, LANES)] = zero

    wqv = [wq_v[pl.ds(cc * LANES, LANES)] for cc in range(dch)]
    sl = slen_v[...]
    lane = lax.iota(jnp.int32, LANES)

    bufs = (buf0, buf1)
    sems = (sem0, sem1)
    handles = [
        pltpu.async_copy(
            x_hbm.at[b, pl.ds(row0 + g * SC_CHUNK, SC_CHUNK), :],
            bufs[g], sems[g])
        for g in range(2)
    ]

    for g in range(nchunk):
        buf = bufs[g % 2]
        handles[g % 2].wait()

        def row_body(rr, _, buf=buf, g=g):
            r_glob = row0 + g * SC_CHUNK + rr
            dot = zero
            for cc in range(dch):
                dot = dot + buf[rr, pl.ds(cc * LANES, LANES)] * wqv[cc]
            w = _lane_sum_splat(dot)
            r_vec = jnp.full((LANES,), r_glob, dtype=jnp.int32)
            segv = _lane_sum_splat((sl <= r_vec).astype(jnp.int32))
            segbase = segv * d + lane
            for cc in range(dch):
                plsc.addupdate_scatter(
                    acc, [segbase + cc * LANES],
                    w * buf[rr, pl.ds(cc * LANES, LANES)])
            return 0

        lax.fori_loop(0, SC_CHUNK, row_body, 0)

        if g + 2 < nchunk:
            handles[g % 2] = pltpu.async_copy(
                x_hbm.at[b, pl.ds(row0 + (g + 2) * SC_CHUNK, SC_CHUNK), :],
                buf, sems[g % 2])

    wid = s * NC + c
    pltpu.sync_copy(acc, g_hbm.at[wid])


def kernel(lengths, nodes_output, Wq):
    B, T, D = nodes_output.shape
    n_len = lengths.shape[0]
    nseg = n_len + 1
    lens = jnp.asarray(lengths, dtype=jnp.int32)
    slen = jnp.sort(lens)
    rank = jnp.argsort(jnp.argsort(lens))

    mesh = plsc.VectorSubcoreMesh(
        core_axis_name="c", subcore_axis_name="s",
        num_cores=NC, num_subcores=NS)
    g_out = pl.kernel(
        functools.partial(_sc_body, d=D, nseg=nseg),
        out_type=jax.ShapeDtypeStruct((NC * NS, nseg * D), jnp.float32),
        mesh=mesh,
        scratch_types=[
            pltpu.VMEM((SC_CHUNK, D), jnp.float32),
            pltpu.VMEM((SC_CHUNK, D), jnp.float32),
            pltpu.VMEM((nseg * D,), jnp.float32),
            pltpu.VMEM((n_len,), jnp.int32),
            pltpu.VMEM((D,), jnp.float32),
            pltpu.SemaphoreType.DMA,
            pltpu.SemaphoreType.DMA,
        ],
        compiler_params=pltpu.CompilerParams(needs_layout_passes=False),
    )(slen, nodes_output, Wq.reshape(D))

    grid = (B, (T - T_SC) // TC_CHUNK)
    off = T_SC // TC_CHUNK
    out_tc = pl.pallas_call(
        functools.partial(_tc_kernel, t_chunk=TC_CHUNK, t0_base=T_SC),
        grid=grid,
        in_specs=[
            pl.BlockSpec((n_len, 1), lambda bb, kt: (0, 0)),
            pl.BlockSpec((1, TC_CHUNK, D), lambda bb, kt: (bb, kt + off, 0)),
            pl.BlockSpec((D, 1), lambda bb, kt: (0, 0)),
        ],
        out_specs=pl.BlockSpec((1, n_len, D), lambda bb, kt: (bb, 0, 0)),
        out_shape=jax.ShapeDtypeStruct((B, n_len, D), jnp.float32),
    )(lens.reshape(n_len, 1), nodes_output, Wq)

    g_b = g_out.reshape(NS, NC, nseg, D).sum(axis=1)
    cum = jnp.cumsum(g_b, axis=1)
    out_sc = cum[:, rank, :]

    out = out_tc + out_sc
    return out.transpose(1, 0, 2).reshape(n_len * B, D)


# speedup vs baseline: 1.6375x; 1.6375x over previous
import functools

import jax
import jax.numpy as jnp
from jax.experimental import pallas as pl


def _agg_kernel(len_ref, x_ref, wq_ref, out_ref, *, t_chunk: int):
    kt = pl.program_id(1)

    @pl.when(kt == 0)
    def _init():
        out_ref[...] = jnp.zeros_like(out_ref)

    xb = x_ref[0]
    d = xb.shape[1]
    w = jnp.sum(xb * wq_ref[...].reshape(1, d), axis=1)

    t0 = kt * t_chunk
    t_idx = jax.lax.broadcasted_iota(jnp.int32, (1, t_chunk), 1) + t0
    mask = (t_idx < len_ref[...]).astype(jnp.float32)
    a = mask * w.reshape(1, t_chunk)

    out_ref[0] += jax.lax.dot_general(
        a, xb, (((1,), (0,)), ((), ())),
        preferred_element_type=jnp.float32)


def kernel(lengths, nodes_output, Wq):
    B, T, D = nodes_output.shape
    n_len = lengths.shape[0]
    t_chunk = 1024
    lens = jnp.asarray(lengths, dtype=jnp.int32).reshape(n_len, 1)

    grid = (B, T // t_chunk)
    out = pl.pallas_call(
        functools.partial(_agg_kernel, t_chunk=t_chunk),
        grid=grid,
        in_specs=[
            pl.BlockSpec((n_len, 1), lambda b, kt: (0, 0)),
            pl.BlockSpec((1, t_chunk, D), lambda b, kt: (b, kt, 0)),
            pl.BlockSpec((D, 1), lambda b, kt: (0, 0)),
        ],
        out_specs=pl.BlockSpec((1, n_len, D), lambda b, kt: (b, 0, 0)),
        out_shape=jax.ShapeDtypeStruct((B, n_len, D), jnp.float32),
    )(lens, nodes_output, Wq)
    return out.transpose(1, 0, 2).reshape(n_len * B, D)


# speedup vs baseline: 2.2623x; 1.3816x over previous
import functools

import jax
import jax.numpy as jnp
from jax.experimental import pallas as pl


def _agg_kernel(len_ref, x_ref, wq_ref, out_ref, *, t_chunk: int):
    kt = pl.program_id(1)

    @pl.when(kt == 0)
    def _init():
        out_ref[...] = jnp.zeros_like(out_ref)

    xb = x_ref[0]
    d = xb.shape[1]
    w = jnp.sum(xb * wq_ref[...].reshape(1, d), axis=1)

    t0 = kt * t_chunk
    t_idx = jax.lax.broadcasted_iota(jnp.int32, (1, t_chunk), 1) + t0
    mask = (t_idx < len_ref[...]).astype(jnp.float32)
    a = mask * w.reshape(1, t_chunk)

    out_ref[0] += jax.lax.dot_general(
        a, xb, (((1,), (0,)), ((), ())),
        preferred_element_type=jnp.float32)


def kernel(lengths, nodes_output, Wq):
    B, T, D = nodes_output.shape
    n_len = lengths.shape[0]
    t_chunk = 2048
    lens = jnp.asarray(lengths, dtype=jnp.int32).reshape(n_len, 1)

    grid = (B, T // t_chunk)
    out = pl.pallas_call(
        functools.partial(_agg_kernel, t_chunk=t_chunk),
        grid=grid,
        in_specs=[
            pl.BlockSpec((n_len, 1), lambda b, kt: (0, 0)),
            pl.BlockSpec((1, t_chunk, D), lambda b, kt: (b, kt, 0)),
            pl.BlockSpec((D, 1), lambda b, kt: (0, 0)),
        ],
        out_specs=pl.BlockSpec((1, n_len, D), lambda b, kt: (b, 0, 0)),
        out_shape=jax.ShapeDtypeStruct((B, n_len, D), jnp.float32),
    )(lens, nodes_output, Wq)
    return out.transpose(1, 0, 2).reshape(n_len * B, D)


# speedup vs baseline: 2.6717x; 1.1810x over previous
import functools

import jax
import jax.numpy as jnp
from jax.experimental import pallas as pl


def _agg_kernel(len_ref, x_ref, wq_ref, out_ref, *, t_chunk: int):
    kt = pl.program_id(1)

    @pl.when(kt == 0)
    def _init():
        out_ref[...] = jnp.zeros_like(out_ref)

    xb = x_ref[0]
    d = xb.shape[1]
    w = jnp.sum(xb * wq_ref[...].reshape(1, d), axis=1)

    t0 = kt * t_chunk
    t_idx = jax.lax.broadcasted_iota(jnp.int32, (1, t_chunk), 1) + t0
    mask = (t_idx < len_ref[...]).astype(jnp.float32)
    a = mask * w.reshape(1, t_chunk)

    out_ref[0] += jax.lax.dot_general(
        a, xb, (((1,), (0,)), ((), ())),
        preferred_element_type=jnp.float32)


def kernel(lengths, nodes_output, Wq):
    B, T, D = nodes_output.shape
    n_len = lengths.shape[0]
    t_chunk = 4096
    lens = jnp.asarray(lengths, dtype=jnp.int32).reshape(n_len, 1)

    grid = (B, T // t_chunk)
    out = pl.pallas_call(
        functools.partial(_agg_kernel, t_chunk=t_chunk),
        grid=grid,
        in_specs=[
            pl.BlockSpec((n_len, 1), lambda b, kt: (0, 0)),
            pl.BlockSpec((1, t_chunk, D), lambda b, kt: (b, kt, 0)),
            pl.BlockSpec((D, 1), lambda b, kt: (0, 0)),
        ],
        out_specs=pl.BlockSpec((1, n_len, D), lambda b, kt: (b, 0, 0)),
        out_shape=jax.ShapeDtypeStruct((B, n_len, D), jnp.float32),
    )(lens, nodes_output, Wq)
    return out.transpose(1, 0, 2).reshape(n_len * B, D)
